# baseline (speedup 1.0000x reference)
import numpy as np

import jax
import jax.numpy as jnp
from jax.experimental import pallas as pl
from jax.experimental.pallas import tpu as pltpu


def _repl_matrix(c_out, T):
    r = np.zeros((c_out * T, c_out), np.float32)
    for i in range(c_out * T):
        r[i, i // T] = 1.0
    return r


def _band_masks(Kt, T, CT):
    m = np.zeros((Kt, T, CT), np.float32)
    for k in range(Kt):
        for tp in range(T):
            if tp + k < T:
                for c in range(tp + k, CT, T):
                    m[k, tp, c] = 1.0
    return m


def _make_body(b_tile, Kt, c_in, c_out, T, T_out, V):
    CT = c_in * T

    def _body(x_ref, lw_ref, w3_ref, ro_ref, ri_ref, m_ref, cb_ref, lb_ref,
              o_ref):
        lw = lw_ref[...]
        lwb = lw.astype(jnp.bfloat16)
        ro = ro_ref[...]
        ri = ri_ref[...]
        acc = None
        for kt in range(Kt):
            a1 = jax.lax.dot_general(
                ro, w3_ref[kt].astype(jnp.bfloat16),
                dimension_numbers=(((1,), (0,)), ((), ())),
                preferred_element_type=jnp.float32)
            a2 = jax.lax.dot_general(
                a1.astype(jnp.bfloat16), ri,
                dimension_numbers=(((1,), (1,)), ((), ())),
                preferred_element_type=jnp.float32)
            mk = jnp.concatenate([m_ref[kt]] * c_out, axis=0)
            a2 = a2 * mk
            acc = a2 if acc is None else acc + a2
        wb = acc.astype(jnp.bfloat16)
        s_col = jax.lax.dot_general(
            lw, jnp.ones((V, 1), jnp.float32),
            dimension_numbers=(((1,), (0,)), ((), ())),
            preferred_element_type=jnp.float32)
        bias = jax.lax.dot_general(
            cb_ref[...], s_col,
            dimension_numbers=(((0,), (1,)), ((), ())),
            preferred_element_type=jnp.float32) + lb_ref[...]
        bias_all = jnp.concatenate([bias] * b_tile, axis=1)
        xws = []
        for b in range(b_tile):
            xb = x_ref[b].reshape(CT, V).astype(jnp.bfloat16)
            xw = jax.lax.dot_general(
                xb, lwb,
                dimension_numbers=(((1,), (1,)), ((), ())),
                preferred_element_type=jnp.float32)
            xws.append(xw.astype(jnp.bfloat16))
        xw_all = jnp.concatenate(xws, axis=1)
        y_all = jax.lax.dot_general(
            wb, xw_all,
            dimension_numbers=(((1,), (0,)), ((), ())),
            preferred_element_type=jnp.float32)
        y3 = y_all.reshape(c_out, T, b_tile * V) + bias_all[:, None, :]
        for b in range(b_tile):
            o_ref[b] = y3[:, :T_out, b * V:(b + 1) * V]
    return _body


def kernel(x, conv_w, conv_b, lin_w, lin_b):
    B, c_in, T, V = x.shape
    c_out, _, Kt, _ = conv_w.shape
    T_out = T - Kt + 1
    CT = c_in * T

    w3 = jnp.transpose(conv_w[:, :, :, 0], (2, 0, 1))
    rmat_o = jnp.asarray(_repl_matrix(c_out, T), jnp.bfloat16)
    rmat_i = jnp.asarray(_repl_matrix(c_in, T), jnp.bfloat16)
    masks = jnp.asarray(_band_masks(Kt, T, CT), jnp.float32)
    cb = conv_b.reshape(1, c_out)
    lb = lin_b.reshape(1, V)

    b_tile = 16
    while B % b_tile:
        b_tile //= 2
    grid = (B // b_tile,)

    return pl.pallas_call(
        _make_body(b_tile, Kt, c_in, c_out, T, T_out, V),
        out_shape=jax.ShapeDtypeStruct((B, c_out, T_out, V), jnp.float32),
        grid=grid,
        in_specs=[
            pl.BlockSpec((b_tile, c_in, T, V), lambda g: (g, 0, 0, 0)),
            pl.BlockSpec((V, V), lambda g: (0, 0)),
            pl.BlockSpec((Kt, c_out, c_in), lambda g: (0, 0, 0)),
            pl.BlockSpec((c_out * T, c_out), lambda g: (0, 0)),
            pl.BlockSpec((c_in * T, c_in), lambda g: (0, 0)),
            pl.BlockSpec((Kt, T, CT), lambda g: (0, 0, 0)),
            pl.BlockSpec((1, c_out), lambda g: (0, 0)),
            pl.BlockSpec((1, V), lambda g: (0, 0)),
        ],
        out_specs=pl.BlockSpec((b_tile, c_out, T_out, V),
                               lambda g: (g, 0, 0, 0)),
        compiler_params=pltpu.CompilerParams(
            dimension_semantics=("parallel",),
            vmem_limit_bytes=64 * 1024 * 1024),
    )(x, lin_w, w3, rmat_o, rmat_i, masks, cb, lb)
